# baseline (speedup 1.0000x reference)
import functools

import jax
import jax.numpy as jnp
from jax import lax
from jax.experimental import pallas as pl
from jax.experimental.pallas import tpu as pltpu

_K = 100


def _stats_kernel(x_ref, bv_ref, s_ref, npos_ref, *, hw):
    def pass_a(k, carry):
        m, s, npos = carry
        xk = x_ref[k]
        return (
            jnp.maximum(m, xk),
            s + xk,
            npos + jnp.where(xk > 0.0, 1.0, 0.0),
        )

    x0 = x_ref[0]
    m, s, npos = lax.fori_loop(
        1,
        hw,
        pass_a,
        (x0, x0, jnp.where(x0 > 0.0, 1.0, 0.0)),
    )

    def pass_b(k, e):
        return e + jnp.exp(x_ref[k] - m)

    e = lax.fori_loop(1, hw, pass_b, jnp.exp(x0 - m))

    bv_ref[...] = m + jnp.log(e) - s * (1.0 / hw)
    s_ref[...] = s
    npos_ref[...] = npos


def _select_kernel(bv_ref, s_ref, npos_ref, y_ref, loss_ref, np_ref, *, b, c):
    y_row = y_ref[...]
    row = lax.broadcasted_iota(jnp.int32, (c, b), 0)
    is_y = row == y_row

    s = s_ref[...] + 0.0
    s = jnp.where(is_y, -jnp.inf, s)

    bits = pltpu.bitcast(s, jnp.int32)
    key = jnp.where(bits < 0, bits ^ jnp.int32(0x7FFFFFFF), bits)

    lo0 = jnp.full((1, b), jnp.int32(-(2**31)), jnp.int32)
    hi0 = jnp.full((1, b), jnp.int32(2**31 - 1), jnp.int32)

    def body(_, carry):
        lo, hi = carry
        mid = (lo >> 1) + (hi >> 1) + (lo & hi & 1)
        cnt = jnp.sum((key >= mid).astype(jnp.int32), axis=0, keepdims=True)
        ok = cnt >= _K
        return jnp.where(ok, mid, lo), jnp.where(ok, hi, mid)

    lo, _ = lax.fori_loop(0, 32, body, (lo0, hi0))
    thr = lo

    gt = key > thr
    eq = key == thr
    need = (_K - jnp.sum(gt.astype(jnp.int32), axis=0, keepdims=True)).astype(
        jnp.float32
    )
    eq_f = eq.astype(jnp.float32)
    r = lax.broadcasted_iota(jnp.int32, (c, c), 0)
    cc = lax.broadcasted_iota(jnp.int32, (c, c), 1)
    tri = (cc < r).astype(jnp.float32)
    rank = lax.dot_general(
        tri, eq_f, (((1,), (0,)), ((), ())), preferred_element_type=jnp.float32
    )
    sel = gt | (eq & (rank < need))

    bv = bv_ref[...]
    loss_ref[...] = jnp.sum(jnp.where(sel, bv, 0.0), keepdims=True).reshape(1, 1) * (
        1.0 / b
    )

    npos = npos_ref[...]
    col_npos = jnp.sum(npos, axis=0, keepdims=True) - jnp.sum(
        jnp.where(is_y, npos, 0.0), axis=0, keepdims=True
    )
    np_ref[...] = jnp.sum(col_npos.astype(jnp.int32), keepdims=True).reshape(1, 1)


@jax.jit
def kernel(x, y):
    B, C, H, W = x.shape
    HW = H * W
    x_t = jnp.transpose(x, (2, 3, 1, 0)).reshape(HW, C, B)

    CB = 200
    bv, s, npos = pl.pallas_call(
        functools.partial(_stats_kernel, hw=HW),
        grid=(C // CB,),
        in_specs=[pl.BlockSpec((HW, CB, B), lambda j: (0, j, 0))],
        out_specs=[
            pl.BlockSpec((CB, B), lambda j: (j, 0)),
            pl.BlockSpec((CB, B), lambda j: (j, 0)),
            pl.BlockSpec((CB, B), lambda j: (j, 0)),
        ],
        out_shape=[
            jax.ShapeDtypeStruct((C, B), jnp.float32),
            jax.ShapeDtypeStruct((C, B), jnp.float32),
            jax.ShapeDtypeStruct((C, B), jnp.float32),
        ],
    )(x_t)

    y2 = y.astype(jnp.int32).reshape(1, B)

    loss, num_posi = pl.pallas_call(
        functools.partial(_select_kernel, b=B, c=C),
        in_specs=[
            pl.BlockSpec((C, B), lambda: (0, 0)),
            pl.BlockSpec((C, B), lambda: (0, 0)),
            pl.BlockSpec((C, B), lambda: (0, 0)),
            pl.BlockSpec((1, B), lambda: (0, 0)),
        ],
        out_specs=[
            pl.BlockSpec((1, 1), lambda: (0, 0)),
            pl.BlockSpec((1, 1), lambda: (0, 0)),
        ],
        out_shape=[
            jax.ShapeDtypeStruct((1, 1), jnp.float32),
            jax.ShapeDtypeStruct((1, 1), jnp.int32),
        ],
    )(bv, s, npos, y2)

    return (loss[0, 0], num_posi[0, 0])


# speedup vs baseline: 1.1609x; 1.1609x over previous
import functools

import jax
import jax.numpy as jnp
from jax import lax
from jax.experimental import pallas as pl
from jax.experimental.pallas import tpu as pltpu
from jax.experimental.pallas import tpu_sc as plsc

_K = 100


def _stats_kernel(x_ref, bv_ref, s_ref, npos_ref, *, hw):
    xb = x_ref[...]
    m = jnp.max(xb, axis=0)
    e = jnp.sum(jnp.exp(xb - m[None]), axis=0)
    s = jnp.sum(xb, axis=0)
    npos = jnp.sum((xb > 0.0).astype(jnp.float32), axis=0)
    bv_ref[...] = m + jnp.log(e) - s * (1.0 / hw)
    s_ref[...] = s
    npos_ref[...] = npos


def _select_kernel(bv_ref, s_ref, y_ref, loss_ref, *, b, c):
    y_row = y_ref[...]
    row = lax.broadcasted_iota(jnp.int32, (c, b), 0)
    is_y = row == y_row

    s = s_ref[...] + 0.0
    s = jnp.where(is_y, -jnp.inf, s)

    bits = pltpu.bitcast(s, jnp.int32)
    key = jnp.where(bits < 0, bits ^ jnp.int32(0x7FFFFFFF), bits)

    lo0 = jnp.full((1, b), jnp.int32(-(2**31)), jnp.int32)
    hi0 = jnp.full((1, b), jnp.int32(2**31 - 1), jnp.int32)

    def body(_, carry):
        lo, hi = carry
        mid = (lo >> 1) + (hi >> 1) + (lo & hi & 1)
        cnt = jnp.sum((key >= mid).astype(jnp.int32), axis=0, keepdims=True)
        ok = cnt >= _K
        return jnp.where(ok, mid, lo), jnp.where(ok, hi, mid)

    lo, _ = lax.fori_loop(0, 32, body, (lo0, hi0))
    thr = lo

    gt = key > thr
    eq = key == thr
    need = (_K - jnp.sum(gt.astype(jnp.int32), axis=0, keepdims=True)).astype(
        jnp.float32
    )
    eq_f = eq.astype(jnp.float32)
    r = lax.broadcasted_iota(jnp.int32, (c, c), 0)
    cc = lax.broadcasted_iota(jnp.int32, (c, c), 1)
    tri = (cc < r).astype(jnp.float32)
    rank = lax.dot_general(
        tri, eq_f, (((1,), (0,)), ((), ())), preferred_element_type=jnp.float32
    )
    sel = gt | (eq & (rank < need))

    bv = bv_ref[...]
    loss_ref[...] = jnp.sum(jnp.where(sel, bv, 0.0), keepdims=True).reshape(1, 1) * (
        1.0 / b
    )


def _npos_sc_kernel(
    npos_hbm,
    npos_flat_hbm,
    cidx_hbm,
    out_hbm,
    chunk_v,
    acc_v,
    idx_v,
    corr_v,
    shared,
    sum_v,
    out_v,
    sem,
    *,
    rows,
    b,
):
    cid = lax.axis_index("c")
    sid = lax.axis_index("s")

    @pl.when(cid == 0)
    def _work():
        pltpu.sync_copy(npos_hbm.at[pl.ds(sid * rows, rows)], chunk_v)
        for cchunk in range(b // 16):
            acc = chunk_v[0, pl.ds(cchunk * 16, 16)]
            for rrow in range(1, rows):
                acc = acc + chunk_v[rrow, pl.ds(cchunk * 16, 16)]
            acc_v[pl.ds(cchunk * 16, 16)] = acc
        pltpu.sync_copy(acc_v, shared.at[sid])

    plsc.subcore_barrier()

    @pl.when((cid == 0) & (sid == 0))
    def _reduce():
        pltpu.sync_copy(shared, sum_v)
        pltpu.sync_copy(cidx_hbm, idx_v)
        pltpu.async_copy(npos_flat_hbm.at[idx_v], corr_v, sem).wait()
        v_tot = jnp.zeros((16,), jnp.int32)
        for cchunk in range(b // 16):
            acc = sum_v[0, pl.ds(cchunk * 16, 16)]
            for t in range(1, 16):
                acc = acc + sum_v[t, pl.ds(cchunk * 16, 16)]
            acc = acc - corr_v[pl.ds(cchunk * 16, 16)]
            v_tot = v_tot + acc.astype(jnp.int32)
        out_v[...] = v_tot
        pltpu.sync_copy(out_v, out_hbm)


@jax.jit
def kernel(x, y):
    B, C, H, W = x.shape
    HW = H * W
    x_t = jnp.transpose(x, (2, 3, 1, 0)).reshape(HW, C, B)

    CB = 200
    bv, s, npos = pl.pallas_call(
        functools.partial(_stats_kernel, hw=HW),
        grid=(C // CB,),
        in_specs=[pl.BlockSpec((HW, CB, B), lambda j: (0, j, 0))],
        out_specs=[
            pl.BlockSpec((CB, B), lambda j: (j, 0)),
            pl.BlockSpec((CB, B), lambda j: (j, 0)),
            pl.BlockSpec((CB, B), lambda j: (j, 0)),
        ],
        out_shape=[
            jax.ShapeDtypeStruct((C, B), jnp.float32),
            jax.ShapeDtypeStruct((C, B), jnp.float32),
            jax.ShapeDtypeStruct((C, B), jnp.float32),
        ],
    )(x_t)

    y2 = y.astype(jnp.int32).reshape(1, B)

    (loss,) = pl.pallas_call(
        functools.partial(_select_kernel, b=B, c=C),
        in_specs=[
            pl.BlockSpec((C, B), lambda: (0, 0)),
            pl.BlockSpec((C, B), lambda: (0, 0)),
            pl.BlockSpec((1, B), lambda: (0, 0)),
        ],
        out_specs=[
            pl.BlockSpec((1, 1), lambda: (0, 0)),
        ],
        out_shape=[
            jax.ShapeDtypeStruct((1, 1), jnp.float32),
        ],
    )(bv, s, y2)

    NS = 16
    CPAD = 1024
    ROWS = CPAD // NS

    npos_pad = jnp.pad(npos, ((0, CPAD - C), (0, 0)))
    npos_flat = npos_pad.reshape(CPAD * B)
    cidx = y.astype(jnp.int32) * B + jnp.arange(B, dtype=jnp.int32)

    mesh = plsc.VectorSubcoreMesh(core_axis_name="c", subcore_axis_name="s")
    npos_fn = functools.partial(
        pl.kernel,
        mesh=mesh,
        out_type=jax.ShapeDtypeStruct((16,), jnp.int32),
        scratch_types=[
            pltpu.VMEM((ROWS, B), jnp.float32),
            pltpu.VMEM((B,), jnp.float32),
            pltpu.VMEM((B,), jnp.int32),
            pltpu.VMEM((B,), jnp.float32),
            pltpu.VMEM_SHARED((NS, B), jnp.float32),
            pltpu.VMEM((NS, B), jnp.float32),
            pltpu.VMEM((16,), jnp.int32),
            pltpu.SemaphoreType.DMA,
        ],
    )(functools.partial(_npos_sc_kernel, rows=ROWS, b=B))
    num_posi = jnp.sum(npos_fn(npos_pad, npos_flat, cidx))

    return (loss[0, 0], num_posi)


# speedup vs baseline: 1.1979x; 1.0318x over previous
import functools

import jax
import jax.numpy as jnp
from jax import lax
from jax.experimental import pallas as pl
from jax.experimental.pallas import tpu as pltpu
from jax.experimental.pallas import tpu_sc as plsc

_K = 100


def _stats_kernel(x_ref, bv_ref, s_ref, npos_ref, part_ref, *, hw):
    xb = x_ref[...]
    m = jnp.max(xb, axis=0)
    e = jnp.sum(jnp.exp(xb - m[None]), axis=0)
    s = jnp.sum(xb, axis=0)
    npos = jnp.sum((xb > 0.0).astype(jnp.float32), axis=0)
    bv_ref[...] = m + jnp.log(e) - s * (1.0 / hw)
    s_ref[...] = s
    npos_ref[...] = npos
    part_ref[0] = jnp.sum(npos, axis=0, keepdims=True)


def _select_kernel(bv_ref, s_ref, y_ref, loss_ref, *, b, c):
    y_row = y_ref[...]
    row = lax.broadcasted_iota(jnp.int32, (c, b), 0)
    is_y = row == y_row

    s = s_ref[...] + 0.0
    s = jnp.where(is_y, -jnp.inf, s)

    bits = pltpu.bitcast(s, jnp.int32)
    key = jnp.where(bits < 0, bits ^ jnp.int32(0x7FFFFFFF), bits)

    lo0 = jnp.full((1, b), jnp.int32(-(2**31)), jnp.int32)
    hi0 = jnp.full((1, b), jnp.int32(2**31 - 1), jnp.int32)

    def body(_, carry):
        lo, hi = carry
        mid = (lo >> 1) + (hi >> 1) + (lo & hi & 1)
        cnt = jnp.sum((key >= mid).astype(jnp.int32), axis=0, keepdims=True)
        ok = cnt >= _K
        return jnp.where(ok, mid, lo), jnp.where(ok, hi, mid)

    lo, _ = lax.fori_loop(0, 32, body, (lo0, hi0))
    thr = lo

    gt = key > thr
    eq = key == thr
    need = (_K - jnp.sum(gt.astype(jnp.int32), axis=0, keepdims=True)).astype(
        jnp.float32
    )
    eq_f = eq.astype(jnp.float32)
    r = lax.broadcasted_iota(jnp.int32, (c, c), 0)
    cc = lax.broadcasted_iota(jnp.int32, (c, c), 1)
    tri = (cc < r).astype(jnp.float32)
    rank = lax.dot_general(
        tri, eq_f, (((1,), (0,)), ((), ())), preferred_element_type=jnp.float32
    )
    sel = gt | (eq & (rank < need))

    bv = bv_ref[...]
    loss_ref[...] = jnp.sum(jnp.where(sel, bv, 0.0), keepdims=True).reshape(1, 1) * (
        1.0 / b
    )


def _npos_sc_kernel(
    part_hbm,
    npos_flat_hbm,
    cidx_hbm,
    out_hbm,
    part_v,
    idx_v,
    corr_v,
    out_v,
    sem,
    *,
    nparts,
    b,
):
    cid = lax.axis_index("c")
    sid = lax.axis_index("s")

    @pl.when((cid == 0) & (sid == 0))
    def _reduce():
        pltpu.sync_copy(part_hbm, part_v)
        pltpu.sync_copy(cidx_hbm, idx_v)
        pltpu.async_copy(npos_flat_hbm.at[idx_v], corr_v, sem).wait()
        v_tot = jnp.zeros((16,), jnp.int32)
        for cchunk in range(b // 16):
            acc = part_v[0, 0, pl.ds(cchunk * 16, 16)]
            for t in range(1, nparts):
                acc = acc + part_v[t, 0, pl.ds(cchunk * 16, 16)]
            acc = acc - corr_v[pl.ds(cchunk * 16, 16)]
            v_tot = v_tot + acc.astype(jnp.int32)
        out_v[...] = v_tot
        pltpu.sync_copy(out_v, out_hbm)


@jax.jit
def kernel(x, y):
    B, C, H, W = x.shape
    HW = H * W
    x_t = jnp.transpose(x, (2, 3, 1, 0)).reshape(HW, C, B)

    CB = 200
    NP = C // CB
    bv, s, npos, parts = pl.pallas_call(
        functools.partial(_stats_kernel, hw=HW),
        grid=(NP,),
        in_specs=[pl.BlockSpec((HW, CB, B), lambda j: (0, j, 0))],
        out_specs=[
            pl.BlockSpec((CB, B), lambda j: (j, 0)),
            pl.BlockSpec((CB, B), lambda j: (j, 0)),
            pl.BlockSpec((CB, B), lambda j: (j, 0)),
            pl.BlockSpec((1, 1, B), lambda j: (j, 0, 0)),
        ],
        out_shape=[
            jax.ShapeDtypeStruct((C, B), jnp.float32),
            jax.ShapeDtypeStruct((C, B), jnp.float32),
            jax.ShapeDtypeStruct((C, B), jnp.float32),
            jax.ShapeDtypeStruct((NP, 1, B), jnp.float32),
        ],
    )(x_t)

    y2 = y.astype(jnp.int32).reshape(1, B)

    (loss,) = pl.pallas_call(
        functools.partial(_select_kernel, b=B, c=C),
        in_specs=[
            pl.BlockSpec((C, B), lambda: (0, 0)),
            pl.BlockSpec((C, B), lambda: (0, 0)),
            pl.BlockSpec((1, B), lambda: (0, 0)),
        ],
        out_specs=[
            pl.BlockSpec((1, 1), lambda: (0, 0)),
        ],
        out_shape=[
            jax.ShapeDtypeStruct((1, 1), jnp.float32),
        ],
    )(bv, s, y2)

    npos_flat = npos.reshape(C * B)
    cidx = y.astype(jnp.int32) * B + jnp.arange(B, dtype=jnp.int32)

    mesh = plsc.VectorSubcoreMesh(core_axis_name="c", subcore_axis_name="s")
    npos_fn = functools.partial(
        pl.kernel,
        mesh=mesh,
        out_type=jax.ShapeDtypeStruct((16,), jnp.int32),
        scratch_types=[
            pltpu.VMEM((NP, 1, B), jnp.float32),
            pltpu.VMEM((B,), jnp.int32),
            pltpu.VMEM((B,), jnp.float32),
            pltpu.VMEM((16,), jnp.int32),
            pltpu.SemaphoreType.DMA,
        ],
    )(functools.partial(_npos_sc_kernel, nparts=NP, b=B))
    num_posi = jnp.sum(npos_fn(parts, npos_flat, cidx))

    return (loss[0, 0], num_posi)


# speedup vs baseline: 1.5943x; 1.3309x over previous
import functools

import jax
import jax.numpy as jnp
from jax import lax
from jax.experimental import pallas as pl
from jax.experimental.pallas import tpu as pltpu

_K = 100


def _stats_kernel(x_ref, bv_ref, s_ref, npos_ref, *, hw):
    xb = x_ref[...]
    m = jnp.max(xb, axis=0)
    e = jnp.sum(jnp.exp(xb - m[None]), axis=0)
    s = jnp.sum(xb, axis=0)
    npos = jnp.sum((xb > 0.0).astype(jnp.float32), axis=0)
    bv_ref[...] = m + jnp.log(e) - s * (1.0 / hw)
    s_ref[...] = s
    npos_ref[...] = npos


def _select_kernel(bv_ref, s_ref, npos_ref, y_ref, loss_ref, np_ref, *, b, c):
    y_row = y_ref[...]
    row = lax.broadcasted_iota(jnp.int32, (c, b), 0)
    is_y = row == y_row

    s = s_ref[...] + 0.0
    s = jnp.where(is_y, -jnp.inf, s)

    bits = pltpu.bitcast(s, jnp.int32)
    key = jnp.where(bits < 0, bits ^ jnp.int32(0x7FFFFFFF), bits)

    lo0 = jnp.full((1, b), jnp.int32(-(2**31)), jnp.int32)
    hi0 = jnp.full((1, b), jnp.int32(2**31 - 1), jnp.int32)

    def body(_, carry):
        lo, hi = carry
        mid = (lo >> 1) + (hi >> 1) + (lo & hi & 1)
        cnt = jnp.sum((key >= mid).astype(jnp.int32), axis=0, keepdims=True)
        ok = cnt >= _K
        return jnp.where(ok, mid, lo), jnp.where(ok, hi, mid)

    lo, _ = lax.fori_loop(0, 32, body, (lo0, hi0))
    thr = lo

    gt = key > thr
    eq = key == thr
    need = (_K - jnp.sum(gt.astype(jnp.int32), axis=0, keepdims=True)).astype(
        jnp.float32
    )
    eq_f = eq.astype(jnp.float32)
    r = lax.broadcasted_iota(jnp.int32, (c, c), 0)
    cc = lax.broadcasted_iota(jnp.int32, (c, c), 1)
    tri = (cc < r).astype(jnp.float32)
    rank = lax.dot_general(
        tri, eq_f, (((1,), (0,)), ((), ())), preferred_element_type=jnp.float32
    )
    sel = gt | (eq & (rank < need))

    bv = bv_ref[...]
    loss_ref[...] = jnp.sum(jnp.where(sel, bv, 0.0), keepdims=True).reshape(1, 1) * (
        1.0 / b
    )

    npos = npos_ref[...]
    col_npos = jnp.sum(npos, axis=0, keepdims=True) - jnp.sum(
        jnp.where(is_y, npos, 0.0), axis=0, keepdims=True
    )
    np_ref[...] = jnp.sum(col_npos.astype(jnp.int32), keepdims=True).reshape(1, 1)


@jax.jit
def kernel(x, y):
    B, C, H, W = x.shape
    HW = H * W
    x_t = jnp.transpose(x, (2, 3, 1, 0)).reshape(HW, C, B)

    CB = 200
    bv, s, npos = pl.pallas_call(
        functools.partial(_stats_kernel, hw=HW),
        grid=(C // CB,),
        in_specs=[pl.BlockSpec((HW, CB, B), lambda j: (0, j, 0))],
        out_specs=[
            pl.BlockSpec((CB, B), lambda j: (j, 0)),
            pl.BlockSpec((CB, B), lambda j: (j, 0)),
            pl.BlockSpec((CB, B), lambda j: (j, 0)),
        ],
        out_shape=[
            jax.ShapeDtypeStruct((C, B), jnp.float32),
            jax.ShapeDtypeStruct((C, B), jnp.float32),
            jax.ShapeDtypeStruct((C, B), jnp.float32),
        ],
    )(x_t)

    y2 = y.astype(jnp.int32).reshape(1, B)

    loss, num_posi = pl.pallas_call(
        functools.partial(_select_kernel, b=B, c=C),
        in_specs=[
            pl.BlockSpec((C, B), lambda: (0, 0)),
            pl.BlockSpec((C, B), lambda: (0, 0)),
            pl.BlockSpec((C, B), lambda: (0, 0)),
            pl.BlockSpec((1, B), lambda: (0, 0)),
        ],
        out_specs=[
            pl.BlockSpec((1, 1), lambda: (0, 0)),
            pl.BlockSpec((1, 1), lambda: (0, 0)),
        ],
        out_shape=[
            jax.ShapeDtypeStruct((1, 1), jnp.float32),
            jax.ShapeDtypeStruct((1, 1), jnp.int32),
        ],
    )(bv, s, npos, y2)

    return (loss[0, 0], num_posi[0, 0])
